# Initial kernel scaffold; baseline (speedup 1.0000x reference)
import jax
import jax.numpy as jnp
from jax.experimental import pallas as pl


def kernel(x, embed_weight, t):
    raise NotImplementedError("write your pallas kernel here")


# speedup vs baseline: 1.4439x; 1.4439x over previous
import functools

import jax
import jax.numpy as jnp
from jax import lax
from jax.experimental import pallas as pl
from jax.experimental.pallas import tpu as pltpu
from jax.experimental.pallas import tpu_sc as plsc

_MAX_SEQ_LEN = 8192
_D_MODEL = 2048

_NC = 2
_NS = 16
_NW = _NC * _NS
_ROWS_PER_W = _MAX_SEQ_LEN // _NW
_CHUNK = 16
_NCHUNK = _ROWS_PER_W // _CHUNK


def _gather_body(idx_hbm, table_hbm, out_hbm, idx_v, buf, sem):
    c = lax.axis_index("c")
    s = lax.axis_index("s")
    wid = s * _NC + c
    base = wid * _ROWS_PER_W
    pltpu.sync_copy(idx_hbm.at[pl.ds(base, _ROWS_PER_W)], idx_v)

    def chunk(i, carry):
        row0 = i * _CHUNK
        pltpu.async_copy(
            table_hbm.at[idx_v.at[pl.ds(row0, _CHUNK)]], buf, sem
        ).wait()
        pltpu.sync_copy(buf, out_hbm.at[pl.ds(base + row0, _CHUNK)])
        return carry

    lax.fori_loop(0, _NCHUNK, chunk, 0)


@jax.jit
def _sc_gather(idx, table):
    mesh = plsc.VectorSubcoreMesh(core_axis_name="c", subcore_axis_name="s")
    return pl.kernel(
        _gather_body,
        out_type=jax.ShapeDtypeStruct((_MAX_SEQ_LEN, _D_MODEL), jnp.float32),
        mesh=mesh,
        scratch_types=[
            pltpu.VMEM((_ROWS_PER_W,), jnp.int32),
            pltpu.VMEM((_CHUNK, _D_MODEL), jnp.float32),
            pltpu.SemaphoreType.DMA,
        ],
    )(idx, table)


def kernel(x, embed_weight, t):
    del x
    pos = jnp.arange(_MAX_SEQ_LEN, dtype=jnp.int32) + jnp.asarray(t, jnp.int32)
    idx = jnp.clip(pos, 0, _MAX_SEQ_LEN - 1)
    return _sc_gather(idx, embed_weight)[None]


# speedup vs baseline: 1.6322x; 1.1304x over previous
import jax
import jax.numpy as jnp
from jax import lax
from jax.experimental import pallas as pl
from jax.experimental.pallas import tpu as pltpu
from jax.experimental.pallas import tpu_sc as plsc

_MAX_SEQ_LEN = 8192
_D_MODEL = 2048

_NC = 2
_NS = 16
_NW = _NC * _NS
_ROWS_PER_W = _MAX_SEQ_LEN // _NW
_CHUNK = 16
_NCHUNK = _ROWS_PER_W // _CHUNK
_NBUF = 2


def _gather_body(idx_hbm, table_hbm, out_hbm,
                 idx_v, buf0, buf1, gsem0, gsem1, ssem0, ssem1):
    wid = lax.axis_index("s") * _NC + lax.axis_index("c")
    base = wid * _ROWS_PER_W
    pltpu.sync_copy(idx_hbm.at[pl.ds(base, _ROWS_PER_W)], idx_v)

    bufs = [buf0, buf1]
    gsems = [gsem0, gsem1]
    ssems = [ssem0, ssem1]

    def gather(g):
        b = g % _NBUF
        return pltpu.async_copy(
            table_hbm.at[idx_v.at[pl.ds(g * _CHUNK, _CHUNK)]],
            bufs[b], gsems[b])

    def scatter(g):
        b = g % _NBUF
        return pltpu.async_copy(
            bufs[b], out_hbm.at[pl.ds(base + g * _CHUNK, _CHUNK)], ssems[b])

    gh = {0: gather(0)}
    sh = {}
    for g in range(_NCHUNK):
        if g >= 1:
            sh[g - 1].wait()
        if g + 1 < _NCHUNK:
            gh[g + 1] = gather(g + 1)
        gh[g].wait()
        sh[g] = scatter(g)
    sh[_NCHUNK - 1].wait()


@jax.jit
def _sc_gather(idx, table):
    mesh = plsc.VectorSubcoreMesh(core_axis_name="c", subcore_axis_name="s")
    return pl.kernel(
        _gather_body,
        out_type=jax.ShapeDtypeStruct((_MAX_SEQ_LEN, _D_MODEL), jnp.float32),
        mesh=mesh,
        scratch_types=[
            pltpu.VMEM((_ROWS_PER_W,), jnp.int32),
            pltpu.VMEM((_CHUNK, _D_MODEL), jnp.float32),
            pltpu.VMEM((_CHUNK, _D_MODEL), jnp.float32),
            pltpu.SemaphoreType.DMA,
            pltpu.SemaphoreType.DMA,
            pltpu.SemaphoreType.DMA,
            pltpu.SemaphoreType.DMA,
        ],
    )(idx, table)


def kernel(x, embed_weight, t):
    del x
    pos = jnp.arange(_MAX_SEQ_LEN, dtype=jnp.int32) + jnp.asarray(t, jnp.int32)
    idx = jnp.clip(pos, 0, _MAX_SEQ_LEN - 1)
    return _sc_gather(idx, embed_weight)[None]
